# baseline (speedup 1.0000x reference)
import functools

import jax
import jax.numpy as jnp
from jax import lax
from jax.experimental import pallas as pl
from jax.experimental.pallas import tpu as pltpu
from jax.experimental.pallas import tpu_sc as plsc

N_SEG = 512
TILE = 8192
NSC = 16
NCORES = 2
LANES = 16
ACC = N_SEG + LANES


def _tc_body(x_ref, el_ref, w1t_ref, b1_ref, w2t_ref, b2_ref, e_ref):
    x = x_ref[...]
    h = jnp.dot(w1t_ref[...], x, preferred_element_type=jnp.float32)
    t = jnp.tanh(h + b1_ref[...])
    ev = jnp.dot(w2t_ref[...], t, preferred_element_type=jnp.float32)
    el = el_ref[...]
    dress = (
        jnp.where(el == 1, jnp.float32(-0.5), jnp.float32(0.0))
        + jnp.where(el == 8, jnp.float32(-75.0), jnp.float32(0.0))
    )
    e_ref[...] = jnp.reshape(ev + b2_ref[...], (TILE,)) + dress


def _per_atom_energy(coord, elems1, W1t, b1c, W2t, b2c, n_pad):
    hidden = W1t.shape[0]
    grid = n_pad // TILE
    out = pl.pallas_call(
        _tc_body,
        grid=(grid,),
        in_specs=[
            pl.BlockSpec((3, TILE), lambda i: (0, i)),
            pl.BlockSpec((TILE,), lambda i: (i,)),
            pl.BlockSpec((hidden, 3), lambda i: (0, 0)),
            pl.BlockSpec((hidden, 1), lambda i: (0, 0)),
            pl.BlockSpec((1, hidden), lambda i: (0, 0)),
            pl.BlockSpec((1, 1), lambda i: (0, 0)),
        ],
        out_specs=pl.BlockSpec((TILE,), lambda i: (i,)),
        out_shape=jax.ShapeDtypeStruct((n_pad,), jnp.float32),
    )(coord, elems1, W1t, b1c, W2t, b2c)
    return out


def _sc_body(e_hbm, ids_hbm, out_hbm, e_v, ids_v, acc_v, shared, big_v, tot_v):
    chunk = e_v.shape[0]
    cid = lax.axis_index("c")
    wid = lax.axis_index("s")
    base = (wid * NCORES + cid) * chunk
    pltpu.sync_copy(e_hbm.at[pl.ds(base, chunk)], e_v)
    pltpu.sync_copy(ids_hbm.at[pl.ds(base, chunk)], ids_v.at[pl.ds(0, chunk)])

    zero = jnp.zeros((LANES,), jnp.float32)
    for j in range(ACC // LANES):
        acc_v[pl.ds(j * LANES, LANES)] = zero

    iota = lax.iota(jnp.int32, LANES)
    last = LANES - 1

    UNROLL = 8

    def scatter_step(j, carry):
        for u in range(UNROLL):
            off = (j * UNROLL + u) * LANES
            idc = ids_v[pl.ds(off, LANES)]
            idn = ids_v[pl.ds(off + 1, LANES)]
            val = e_v[pl.ds(off, LANES)]
            p = plsc.cumsum(val)
            bnd = (idc != idn) & (iota < last)
            end_mask = bnd | (iota == last)
            plsc.addupdate_scatter(acc_v, [idc], p, mask=end_mask)
            plsc.addupdate_scatter(acc_v, [idn], -p, mask=bnd)
        return carry

    lax.fori_loop(0, chunk // (LANES * UNROLL), scatter_step, 0)

    pltpu.sync_copy(acc_v, shared.at[pl.ds(wid * ACC, ACC)])
    plsc.subcore_barrier()

    @pl.when(wid == 0)
    def _():
        pltpu.sync_copy(shared, big_v)
        for ch in range(N_SEG // LANES):
            s16 = big_v[pl.ds(ch * LANES, LANES)]
            for r in range(1, NSC):
                s16 = s16 + big_v[pl.ds(r * ACC + ch * LANES, LANES)]
            tot_v[pl.ds(ch * LANES, LANES)] = s16
        pltpu.sync_copy(tot_v, out_hbm.at[pl.ds(cid * N_SEG, N_SEG)])


def _segment_sum_sc(e_p, ids_p):
    n_pad = e_p.shape[0]
    chunk = n_pad // (NSC * NCORES)
    mesh = plsc.VectorSubcoreMesh(
        core_axis_name="c", subcore_axis_name="s", num_cores=NCORES
    )
    run = functools.partial(
        pl.kernel,
        out_type=jax.ShapeDtypeStruct((NCORES * N_SEG,), jnp.float32),
        mesh=mesh,
        compiler_params=pltpu.CompilerParams(needs_layout_passes=False),
        scratch_types=[
            pltpu.VMEM((chunk,), jnp.float32),
            pltpu.VMEM((chunk + LANES,), jnp.int32),
            pltpu.VMEM((ACC,), jnp.float32),
            pltpu.VMEM_SHARED((NSC * ACC,), jnp.float32),
            pltpu.VMEM((NSC * ACC,), jnp.float32),
            pltpu.VMEM((N_SEG,), jnp.float32),
        ],
    )(_sc_body)
    return run(e_p, ids_p)


def kernel(ind_1, elems, coord, W1, b1, W2, b2):
    n = coord.shape[0]
    ids = ind_1.reshape(-1).astype(jnp.int32)
    n_pad = -(-n // TILE) * TILE
    pad = n_pad - n
    coordT = coord.T
    elems1 = elems.astype(jnp.int32)
    ids_p = jnp.pad(ids, (0, pad), constant_values=N_SEG)
    e_p = _per_atom_energy(
        coordT, elems1, W1.T, b1.reshape(-1, 1), W2.T, b2.reshape(1, 1), n_pad
    )
    parts = _segment_sum_sc(e_p, ids_p)
    return parts[:N_SEG] + parts[N_SEG:]


# speedup vs baseline: 1.0179x; 1.0179x over previous
import functools

import jax
import jax.numpy as jnp
from jax import lax
from jax.experimental import pallas as pl
from jax.experimental.pallas import tpu as pltpu
from jax.experimental.pallas import tpu_sc as plsc

N_SEG = 512
TILE = 8192
NSC = 16
NCORES = 2
LANES = 16
ACC = N_SEG + LANES


def _tc_body(x_ref, el_ref, w1t_ref, b1_ref, w2t_ref, b2_ref, e_ref):
    x = x_ref[...]
    h = jnp.dot(w1t_ref[...], x, preferred_element_type=jnp.float32)
    t = jnp.tanh(h + b1_ref[...])
    ev = jnp.dot(w2t_ref[...], t, preferred_element_type=jnp.float32)
    el = el_ref[...]
    dress = (
        jnp.where(el == 1, jnp.float32(-0.5), jnp.float32(0.0))
        + jnp.where(el == 8, jnp.float32(-75.0), jnp.float32(0.0))
    )
    e_ref[...] = jnp.reshape(ev + b2_ref[...], (TILE,)) + dress


def _per_atom_energy(coord, elems1, W1t, b1c, W2t, b2c, n_pad):
    hidden = W1t.shape[0]
    grid = n_pad // TILE
    out = pl.pallas_call(
        _tc_body,
        grid=(grid,),
        in_specs=[
            pl.BlockSpec((3, TILE), lambda i: (0, i)),
            pl.BlockSpec((TILE,), lambda i: (i,)),
            pl.BlockSpec((hidden, 3), lambda i: (0, 0)),
            pl.BlockSpec((hidden, 1), lambda i: (0, 0)),
            pl.BlockSpec((1, hidden), lambda i: (0, 0)),
            pl.BlockSpec((1, 1), lambda i: (0, 0)),
        ],
        out_specs=pl.BlockSpec((TILE,), lambda i: (i,)),
        out_shape=jax.ShapeDtypeStruct((n_pad,), jnp.float32),
    )(coord, elems1, W1t, b1c, W2t, b2c)
    return out


def _sc_body(e_hbm, ids_hbm, out_hbm, e_v, ids_v, acc_v, shared, big_v, tot_v):
    chunk = e_v.shape[0]
    wid = lax.axis_index("s")
    base = wid * chunk
    pltpu.sync_copy(e_hbm.at[pl.ds(base, chunk)], e_v)
    pltpu.sync_copy(ids_hbm.at[pl.ds(base, chunk)], ids_v.at[pl.ds(0, chunk)])

    zero = jnp.zeros((LANES,), jnp.float32)
    for j in range(ACC // LANES):
        acc_v[pl.ds(j * LANES, LANES)] = zero

    iota = lax.iota(jnp.int32, LANES)
    last = LANES - 1

    UNROLL = 8

    def scatter_step(j, carry):
        for u in range(UNROLL):
            off = (j * UNROLL + u) * LANES
            idc = ids_v[pl.ds(off, LANES)]
            idn = ids_v[pl.ds(off + 1, LANES)]
            val = e_v[pl.ds(off, LANES)]
            p = plsc.cumsum(val)
            bnd = (idc != idn) & (iota < last)
            end_mask = bnd | (iota == last)
            plsc.addupdate_scatter(acc_v, [idc], p, mask=end_mask)
            plsc.addupdate_scatter(acc_v, [idn], -p, mask=bnd)
        return carry

    lax.fori_loop(0, chunk // (LANES * UNROLL), scatter_step, 0)

    pltpu.sync_copy(acc_v, shared.at[pl.ds(wid * ACC, ACC)])
    plsc.subcore_barrier()

    @pl.when(wid == 0)
    def _():
        pltpu.sync_copy(shared, big_v)
        for ch in range(N_SEG // LANES):
            s16 = big_v[pl.ds(ch * LANES, LANES)]
            for r in range(1, NSC):
                s16 = s16 + big_v[pl.ds(r * ACC + ch * LANES, LANES)]
            tot_v[pl.ds(ch * LANES, LANES)] = s16
        pltpu.sync_copy(tot_v, out_hbm)


def _segment_sum_sc(e_p, ids_p):
    n_pad = e_p.shape[0]
    chunk = n_pad // NSC
    mesh = plsc.VectorSubcoreMesh(
        core_axis_name="c", subcore_axis_name="s", num_cores=1
    )
    run = functools.partial(
        pl.kernel,
        out_type=jax.ShapeDtypeStruct((N_SEG,), jnp.float32),
        mesh=mesh,
        compiler_params=pltpu.CompilerParams(needs_layout_passes=False),
        scratch_types=[
            pltpu.VMEM((chunk,), jnp.float32),
            pltpu.VMEM((chunk + LANES,), jnp.int32),
            pltpu.VMEM((ACC,), jnp.float32),
            pltpu.VMEM_SHARED((NSC * ACC,), jnp.float32),
            pltpu.VMEM((NSC * ACC,), jnp.float32),
            pltpu.VMEM((N_SEG,), jnp.float32),
        ],
    )(_sc_body)
    return run(e_p, ids_p)


def kernel(ind_1, elems, coord, W1, b1, W2, b2):
    n = coord.shape[0]
    ids2 = ind_1[:, :1] if ind_1.ndim == 2 else ind_1.reshape(-1, 1)
    n_pad = -(-n // TILE) * TILE
    pad = n_pad - n
    coordT = coord.T
    elems1 = elems.astype(jnp.int32)
    ids_p = jnp.pad(
        ids2.astype(jnp.int32), ((0, pad), (0, 0)), constant_values=N_SEG
    ).reshape(-1)
    e_p = _per_atom_energy(
        coordT, elems1, W1.T, b1.reshape(-1, 1), W2.T, b2.reshape(1, 1), n_pad
    )
    return _segment_sum_sc(e_p, ids_p)
